# baseline (speedup 1.0000x reference)
import jax
import jax.numpy as jnp
from jax import lax
from jax.experimental import pallas as pl
from jax.experimental.pallas import tpu as pltpu
from jax.experimental.pallas import tpu_sc as plsc

N_NODES = 10000
N_EDGES = 320000
D_FEAT = 128
N_HID = 20
DP = 32

NC = 2
NS = 16
NW = NC * NS
CHUNK = 128
NCHUNK0 = 104
NCHUNK1 = 56
TOTCHUNK = NS * (NCHUNK0 + NCHUNK1)
E_PAD = TOTCHUNK * CHUNK
NCHUNKD = TOTCHUNK // NW
ROWS_PT = N_NODES // NS
S_ROWS = N_NODES + NS
NBUF = 8
LAG = 3
MAXQ = 8

_MESH = plsc.VectorSubcoreMesh(core_axis_name="c", subcore_axis_name="s")
_SC_PARAMS = pltpu.CompilerParams(use_tc_tiling_on_sc=False)


def _guarded(cond, fn):
    if cond is None:
        fn()
    else:
        pl.when(cond)(fn)


def _fill(ref, nrows, ncols, value):
    v = jnp.full((16,), value, jnp.float32)

    def body(r, carry):
        for k in range(ncols // 16):
            ref[r, pl.ds(k * 16, 16)] = v
        return carry

    lax.fori_loop(0, nrows, body, 0)


def _deg_body(cidx_hbm, out_hbm, cidx_v, ones_v, zbuf_v, s_sh, sem):
    c = lax.axis_index("c")
    s = lax.axis_index("s")
    w = c * NS + s
    pltpu.sync_copy(cidx_hbm.at[pl.ds(w * NCHUNKD, NCHUNKD)], cidx_v)
    _fill(ones_v, CHUNK, 16, 1.0)
    _fill(zbuf_v, ROWS_PT, 16, 0.0)
    pltpu.sync_copy(zbuf_v, s_sh.at[pl.ds(s * ROWS_PT, ROWS_PT)])
    plsc.subcore_barrier()

    def step(j, carry):
        pltpu.async_copy(ones_v, s_sh.at[cidx_v.at[j]], sem, add=True)

        @pl.when(j >= MAXQ)
        def _():
            pltpu.make_async_copy(ones_v, s_sh.at[cidx_v.at[j - MAXQ]],
                                  sem).wait()
        return carry

    lax.fori_loop(0, NCHUNKD, step, 0)
    for j in range(NCHUNKD - MAXQ, NCHUNKD):
        pltpu.make_async_copy(ones_v, s_sh.at[cidx_v.at[j]], sem).wait()
    plsc.subcore_barrier()
    pltpu.sync_copy(
        s_sh.at[pl.ds(s * ROWS_PT, ROWS_PT)],
        out_hbm.at[c, s],
    )


_deg_call = pl.kernel(
    _deg_body,
    out_type=jax.ShapeDtypeStruct((NC, NS, ROWS_PT, 16), jnp.float32),
    mesh=_MESH,
    scratch_types=[
        pltpu.VMEM((NCHUNKD, CHUNK), jnp.int32),
        pltpu.VMEM((CHUNK, 16), jnp.float32),
        pltpu.VMEM((ROWS_PT, 16), jnp.float32),
        pltpu.VMEM_SHARED((S_ROWS, 16), jnp.float32),
        pltpu.SemaphoreType.DMA,
    ],
    compiler_params=_SC_PARAMS,
)


def _agg_body(u_hbm, ridx_hbm, cidx_hbm, out_hbm,
              ridx_v, cidx_v, gbuf_v, zbuf_v, s_sh, *sems):
    gsems = sems[:NBUF]
    ssems = sems[NBUF:]
    c = lax.axis_index("c")
    s = lax.axis_index("s")
    base = jnp.where(c == 0, s * NCHUNK0, NS * NCHUNK0 + s * NCHUNK1)
    limit = jnp.where(c == 0, NCHUNK0, NCHUNK1)
    nx = NCHUNK0 - NCHUNK1
    pltpu.sync_copy(ridx_hbm.at[pl.ds(base, NCHUNK1)], ridx_v.at[pl.ds(0, NCHUNK1)])
    pltpu.sync_copy(cidx_hbm.at[pl.ds(base, NCHUNK1)], cidx_v.at[pl.ds(0, NCHUNK1)])
    if nx:
        @pl.when(c == 0)
        def _():
            pltpu.sync_copy(ridx_hbm.at[pl.ds(base + NCHUNK1, nx)],
                            ridx_v.at[pl.ds(NCHUNK1, nx)])
            pltpu.sync_copy(cidx_hbm.at[pl.ds(base + NCHUNK1, nx)],
                            cidx_v.at[pl.ds(NCHUNK1, nx)])

    _fill(zbuf_v, ROWS_PT, DP, 0.0)
    pltpu.sync_copy(zbuf_v, s_sh.at[pl.ds(s * ROWS_PT, ROWS_PT)])
    plsc.subcore_barrier()

    def G(j):
        return None if j < NCHUNK1 else (c == 0)

    def fire_gather(j):
        b = j % NBUF
        pltpu.async_copy(u_hbm.at[ridx_v.at[j]], gbuf_v.at[b], gsems[b])

    def wait_gather(j):
        b = j % NBUF
        pltpu.make_async_copy(u_hbm.at[ridx_v.at[j]], gbuf_v.at[b],
                              gsems[b]).wait()

    def fire_scatter(j):
        b = j % NBUF
        pltpu.async_copy(gbuf_v.at[b], s_sh.at[cidx_v.at[j]], ssems[b],
                         add=True)

    def wait_scatter(j):
        b = j % NBUF
        pltpu.make_async_copy(gbuf_v.at[b], s_sh.at[cidx_v.at[j]],
                              ssems[b]).wait()

    for j in range(min(NBUF, NCHUNK0)):
        _guarded(G(j), lambda j=j: fire_gather(j))
    waited = set()
    for j in range(NCHUNK0):
        def step(j=j):
            wait_gather(j)
            fire_scatter(j)
        _guarded(G(j), step)
        k = j - LAG
        if k >= 0 and k + NBUF < NCHUNK0:
            _guarded(G(k), lambda k=k: wait_scatter(k))
            waited.add(k)
            _guarded(G(k + NBUF), lambda k=k: fire_gather(k + NBUF))
    for k in range(NCHUNK0):
        if k not in waited:
            _guarded(G(k), lambda k=k: wait_scatter(k))
    plsc.subcore_barrier()
    pltpu.sync_copy(
        s_sh.at[pl.ds(s * ROWS_PT, ROWS_PT)],
        out_hbm.at[c, s],
    )


_agg_call = pl.kernel(
    _agg_body,
    out_type=jax.ShapeDtypeStruct((NC, NS, ROWS_PT, DP), jnp.float32),
    mesh=_MESH,
    scratch_types=[
        pltpu.VMEM((NCHUNK0, CHUNK), jnp.int32),
        pltpu.VMEM((NCHUNK0, CHUNK), jnp.int32),
        pltpu.VMEM((NBUF, CHUNK, DP), jnp.float32),
        pltpu.VMEM((ROWS_PT, DP), jnp.float32),
        pltpu.VMEM_SHARED((S_ROWS, DP), jnp.float32),
    ] + [pltpu.SemaphoreType.DMA] * (2 * NBUF),
    compiler_params=_SC_PARAMS,
)

BR = 1000
GRID = N_NODES // BR


def _t1_body(x_ref, w1_ref, degp_ref, dinv_ref, u1_ref):
    a = degp_ref[...]
    deg = a[0, :, 0:1] + a[1, :, 0:1] + 1.0
    dinv = lax.rsqrt(deg)
    h = jnp.dot(x_ref[...], w1_ref[...], preferred_element_type=jnp.float32)
    dinv_ref[...] = dinv
    u1_ref[...] = dinv * h


def _t2_body(u1_ref, sp_ref, dinv_ref, b1_ref, w2_ref, u2_ref):
    sp = sp_ref[...]
    dinv = dinv_ref[...]
    t = dinv * (u1_ref[...] + sp[0] + sp[1]) + b1_ref[...]
    out1 = jnp.maximum(t, 0.0)
    u2_ref[...] = dinv * jnp.dot(out1, w2_ref[...],
                                 preferred_element_type=jnp.float32)


def _t3_body(u2_ref, sp_ref, dinv_ref, b2_ref, out_ref):
    sp = sp_ref[...]
    z = dinv_ref[...] * (u2_ref[...] + sp[0] + sp[1]) + b2_ref[...]
    z = jnp.maximum(z, 0.0)
    col = lax.broadcasted_iota(jnp.int32, z.shape, 1)
    valid = col < N_HID
    zm = jnp.where(valid, z, -jnp.inf)
    m = jnp.max(zm, axis=1, keepdims=True)
    e = jnp.where(valid, jnp.exp(z - m), 0.0)
    lse = m + jnp.log(jnp.sum(e, axis=1, keepdims=True))
    out_ref[...] = z - lse


def _t1_call(x, w1p, degp):
    return pl.pallas_call(
        _t1_body,
        grid=(GRID,),
        in_specs=[
            pl.BlockSpec((BR, D_FEAT), lambda i: (i, 0)),
            pl.BlockSpec((D_FEAT, DP), lambda i: (0, 0)),
            pl.BlockSpec((NC, BR, 16), lambda i: (0, i, 0)),
        ],
        out_specs=[
            pl.BlockSpec((BR, 1), lambda i: (i, 0)),
            pl.BlockSpec((BR, DP), lambda i: (i, 0)),
        ],
        out_shape=[
            jax.ShapeDtypeStruct((N_NODES, 1), jnp.float32),
            jax.ShapeDtypeStruct((N_NODES, DP), jnp.float32),
        ],
    )(x, w1p, degp)


def _t2_call(u1, sp, dinv, b1p, w2p):
    return pl.pallas_call(
        _t2_body,
        grid=(GRID,),
        in_specs=[
            pl.BlockSpec((BR, DP), lambda i: (i, 0)),
            pl.BlockSpec((NC, BR, DP), lambda i: (0, i, 0)),
            pl.BlockSpec((BR, 1), lambda i: (i, 0)),
            pl.BlockSpec((1, DP), lambda i: (0, 0)),
            pl.BlockSpec((DP, DP), lambda i: (0, 0)),
        ],
        out_specs=pl.BlockSpec((BR, DP), lambda i: (i, 0)),
        out_shape=jax.ShapeDtypeStruct((N_NODES, DP), jnp.float32),
    )(u1, sp, dinv, b1p, w2p)


def _t3_call(u2, sp, dinv, b2p):
    return pl.pallas_call(
        _t3_body,
        grid=(GRID,),
        in_specs=[
            pl.BlockSpec((BR, DP), lambda i: (i, 0)),
            pl.BlockSpec((NC, BR, DP), lambda i: (0, i, 0)),
            pl.BlockSpec((BR, 1), lambda i: (i, 0)),
            pl.BlockSpec((1, DP), lambda i: (0, 0)),
        ],
        out_specs=pl.BlockSpec((BR, DP), lambda i: (i, 0)),
        out_shape=jax.ShapeDtypeStruct((N_NODES, DP), jnp.float32),
    )(u2, sp, dinv, b2p)


def kernel(x, edge_index, W1, b1, W2, b2):
    row = edge_index[0].astype(jnp.int32)
    col = edge_index[1].astype(jnp.int32)
    pad = E_PAD - N_EDGES
    rowp = jnp.concatenate([row, jnp.zeros((pad,), jnp.int32)])
    rowp = rowp.reshape(TOTCHUNK, CHUNK)
    colp = jnp.concatenate([col, jnp.full((pad,), N_NODES, jnp.int32)])
    colp = colp.reshape(TOTCHUNK, CHUNK)

    w1p = jnp.pad(W1, ((0, 0), (0, DP - N_HID)))
    w2p = jnp.pad(W2, ((0, DP - N_HID), (0, DP - N_HID)))
    b1p = jnp.pad(b1, (0, DP - N_HID)).reshape(1, DP)
    b2p = jnp.pad(b2, (0, DP - N_HID)).reshape(1, DP)

    degp = _deg_call(colp).reshape(NC, N_NODES, 16)
    dinv, u1 = _t1_call(x, w1p, degp)
    s1 = _agg_call(u1, rowp, colp).reshape(NC, N_NODES, DP)
    u2 = _t2_call(u1, s1, dinv, b1p, w2p)
    s2 = _agg_call(u2, rowp, colp).reshape(NC, N_NODES, DP)
    z = _t3_call(u2, s2, dinv, b2p)
    return z[:, :N_HID]


# speedup vs baseline: 1.0007x; 1.0007x over previous
import jax
import jax.numpy as jnp
from jax import lax
from jax.experimental import pallas as pl
from jax.experimental.pallas import tpu as pltpu
from jax.experimental.pallas import tpu_sc as plsc

N_NODES = 10000
N_EDGES = 320000
D_FEAT = 128
N_HID = 20
DP = 32

NC = 2
NS = 16
NW = NC * NS
CHUNK = 128
NCHUNK0 = 120
NCHUNK1 = 40
TOTCHUNK = NS * (NCHUNK0 + NCHUNK1)
E_PAD = TOTCHUNK * CHUNK
NCHUNKD = TOTCHUNK // NW
ROWS_PT = N_NODES // NS
S_ROWS = N_NODES + NS
NBUF = 8
LAG = 3
MAXQ = 8

_MESH = plsc.VectorSubcoreMesh(core_axis_name="c", subcore_axis_name="s")
_SC_PARAMS = pltpu.CompilerParams(use_tc_tiling_on_sc=False)


def _guarded(cond, fn):
    if cond is None:
        fn()
    else:
        pl.when(cond)(fn)


def _fill(ref, nrows, ncols, value):
    v = jnp.full((16,), value, jnp.float32)

    def body(r, carry):
        for k in range(ncols // 16):
            ref[r, pl.ds(k * 16, 16)] = v
        return carry

    lax.fori_loop(0, nrows, body, 0)


def _deg_body(cidx_hbm, out_hbm, cidx_v, ones_v, zbuf_v, s_sh, sem):
    c = lax.axis_index("c")
    s = lax.axis_index("s")
    w = c * NS + s
    pltpu.sync_copy(cidx_hbm.at[pl.ds(w * NCHUNKD, NCHUNKD)], cidx_v)
    _fill(ones_v, CHUNK, 16, 1.0)
    _fill(zbuf_v, ROWS_PT, 16, 0.0)
    pltpu.sync_copy(zbuf_v, s_sh.at[pl.ds(s * ROWS_PT, ROWS_PT)])
    plsc.subcore_barrier()

    descs = []
    for j in range(NCHUNKD):
        if j >= MAXQ:
            descs[j - MAXQ].wait()
        descs.append(
            pltpu.async_copy(ones_v, s_sh.at[cidx_v.at[j]], sem, add=True)
        )
    for d in descs[NCHUNKD - MAXQ:]:
        d.wait()
    plsc.subcore_barrier()
    pltpu.sync_copy(
        s_sh.at[pl.ds(s * ROWS_PT, ROWS_PT)],
        out_hbm.at[c, s],
    )


_deg_call = pl.kernel(
    _deg_body,
    out_type=jax.ShapeDtypeStruct((NC, NS, ROWS_PT, 16), jnp.float32),
    mesh=_MESH,
    scratch_types=[
        pltpu.VMEM((NCHUNKD, CHUNK), jnp.int32),
        pltpu.VMEM((CHUNK, 16), jnp.float32),
        pltpu.VMEM((ROWS_PT, 16), jnp.float32),
        pltpu.VMEM_SHARED((S_ROWS, 16), jnp.float32),
        pltpu.SemaphoreType.DMA,
    ],
    compiler_params=_SC_PARAMS,
)


def _agg_body(u_hbm, ridx_hbm, cidx_hbm, out_hbm,
              ridx_v, cidx_v, gbuf_v, zbuf_v, s_sh, *sems):
    gsems = sems[:NBUF]
    ssems = sems[NBUF:]
    c = lax.axis_index("c")
    s = lax.axis_index("s")
    base = jnp.where(c == 0, s * NCHUNK0, NS * NCHUNK0 + s * NCHUNK1)
    limit = jnp.where(c == 0, NCHUNK0, NCHUNK1)
    nx = NCHUNK0 - NCHUNK1
    pltpu.sync_copy(ridx_hbm.at[pl.ds(base, NCHUNK1)], ridx_v.at[pl.ds(0, NCHUNK1)])
    pltpu.sync_copy(cidx_hbm.at[pl.ds(base, NCHUNK1)], cidx_v.at[pl.ds(0, NCHUNK1)])
    if nx:
        @pl.when(c == 0)
        def _():
            pltpu.sync_copy(ridx_hbm.at[pl.ds(base + NCHUNK1, nx)],
                            ridx_v.at[pl.ds(NCHUNK1, nx)])
            pltpu.sync_copy(cidx_hbm.at[pl.ds(base + NCHUNK1, nx)],
                            cidx_v.at[pl.ds(NCHUNK1, nx)])

    _fill(zbuf_v, ROWS_PT, DP, 0.0)
    pltpu.sync_copy(zbuf_v, s_sh.at[pl.ds(s * ROWS_PT, ROWS_PT)])
    plsc.subcore_barrier()

    def G(j):
        return None if j < NCHUNK1 else (c == 0)

    def fire_gather(j):
        b = j % NBUF
        pltpu.async_copy(u_hbm.at[ridx_v.at[j]], gbuf_v.at[b], gsems[b])

    def wait_gather(j):
        b = j % NBUF
        pltpu.make_async_copy(u_hbm.at[ridx_v.at[j]], gbuf_v.at[b],
                              gsems[b]).wait()

    def fire_scatter(j):
        b = j % NBUF
        pltpu.async_copy(gbuf_v.at[b], s_sh.at[cidx_v.at[j]], ssems[b],
                         add=True)

    def wait_scatter(j):
        b = j % NBUF
        pltpu.make_async_copy(gbuf_v.at[b], s_sh.at[cidx_v.at[j]],
                              ssems[b]).wait()

    for j in range(min(NBUF, NCHUNK0)):
        _guarded(G(j), lambda j=j: fire_gather(j))
    waited = set()
    for j in range(NCHUNK0):
        def step(j=j):
            wait_gather(j)
            fire_scatter(j)
        _guarded(G(j), step)
        k = j - LAG
        if k >= 0 and k + NBUF < NCHUNK0:
            _guarded(G(k), lambda k=k: wait_scatter(k))
            waited.add(k)
            _guarded(G(k + NBUF), lambda k=k: fire_gather(k + NBUF))
    for k in range(NCHUNK0):
        if k not in waited:
            _guarded(G(k), lambda k=k: wait_scatter(k))
    plsc.subcore_barrier()
    pltpu.sync_copy(
        s_sh.at[pl.ds(s * ROWS_PT, ROWS_PT)],
        out_hbm.at[c, s],
    )


_agg_call = pl.kernel(
    _agg_body,
    out_type=jax.ShapeDtypeStruct((NC, NS, ROWS_PT, DP), jnp.float32),
    mesh=_MESH,
    scratch_types=[
        pltpu.VMEM((NCHUNK0, CHUNK), jnp.int32),
        pltpu.VMEM((NCHUNK0, CHUNK), jnp.int32),
        pltpu.VMEM((NBUF, CHUNK, DP), jnp.float32),
        pltpu.VMEM((ROWS_PT, DP), jnp.float32),
        pltpu.VMEM_SHARED((S_ROWS, DP), jnp.float32),
    ] + [pltpu.SemaphoreType.DMA] * (2 * NBUF),
    compiler_params=_SC_PARAMS,
)

BR = 1000
GRID = N_NODES // BR


def _ta_body(x_ref, w1_ref, h1_ref):
    h1_ref[...] = jnp.dot(x_ref[...], w1_ref[...],
                          preferred_element_type=jnp.float32)


def _tb_body(h1_ref, degp_ref, dinv_ref, u1_ref):
    a = degp_ref[...]
    deg = a[0, :, 0:1] + a[1, :, 0:1] + 1.0
    dinv = lax.rsqrt(deg)
    dinv_ref[...] = dinv
    u1_ref[...] = dinv * h1_ref[...]


def _t2_body(u1_ref, sp_ref, dinv_ref, b1_ref, w2_ref, u2_ref):
    sp = sp_ref[...]
    dinv = dinv_ref[...]
    t = dinv * (u1_ref[...] + sp[0] + sp[1]) + b1_ref[...]
    out1 = jnp.maximum(t, 0.0)
    u2_ref[...] = dinv * jnp.dot(out1, w2_ref[...],
                                 preferred_element_type=jnp.float32)


def _t3_body(u2_ref, sp_ref, dinv_ref, b2_ref, out_ref):
    sp = sp_ref[...]
    z = dinv_ref[...] * (u2_ref[...] + sp[0] + sp[1]) + b2_ref[...]
    z = jnp.maximum(z, 0.0)
    col = lax.broadcasted_iota(jnp.int32, z.shape, 1)
    valid = col < N_HID
    zm = jnp.where(valid, z, -jnp.inf)
    m = jnp.max(zm, axis=1, keepdims=True)
    e = jnp.where(valid, jnp.exp(z - m), 0.0)
    lse = m + jnp.log(jnp.sum(e, axis=1, keepdims=True))
    out_ref[...] = z - lse


def _ta_call(x, w1p):
    return pl.pallas_call(
        _ta_body,
        grid=(GRID,),
        in_specs=[
            pl.BlockSpec((BR, D_FEAT), lambda i: (i, 0)),
            pl.BlockSpec((D_FEAT, DP), lambda i: (0, 0)),
        ],
        out_specs=pl.BlockSpec((BR, DP), lambda i: (i, 0)),
        out_shape=jax.ShapeDtypeStruct((N_NODES, DP), jnp.float32),
    )(x, w1p)


def _tb_call(h1, degp):
    return pl.pallas_call(
        _tb_body,
        grid=(GRID,),
        in_specs=[
            pl.BlockSpec((BR, DP), lambda i: (i, 0)),
            pl.BlockSpec((NC, BR, 16), lambda i: (0, i, 0)),
        ],
        out_specs=[
            pl.BlockSpec((BR, 1), lambda i: (i, 0)),
            pl.BlockSpec((BR, DP), lambda i: (i, 0)),
        ],
        out_shape=[
            jax.ShapeDtypeStruct((N_NODES, 1), jnp.float32),
            jax.ShapeDtypeStruct((N_NODES, DP), jnp.float32),
        ],
    )(h1, degp)


def _t2_call(u1, sp, dinv, b1p, w2p):
    return pl.pallas_call(
        _t2_body,
        grid=(GRID,),
        in_specs=[
            pl.BlockSpec((BR, DP), lambda i: (i, 0)),
            pl.BlockSpec((NC, BR, DP), lambda i: (0, i, 0)),
            pl.BlockSpec((BR, 1), lambda i: (i, 0)),
            pl.BlockSpec((1, DP), lambda i: (0, 0)),
            pl.BlockSpec((DP, DP), lambda i: (0, 0)),
        ],
        out_specs=pl.BlockSpec((BR, DP), lambda i: (i, 0)),
        out_shape=jax.ShapeDtypeStruct((N_NODES, DP), jnp.float32),
    )(u1, sp, dinv, b1p, w2p)


def _t3_call(u2, sp, dinv, b2p):
    return pl.pallas_call(
        _t3_body,
        grid=(GRID,),
        in_specs=[
            pl.BlockSpec((BR, DP), lambda i: (i, 0)),
            pl.BlockSpec((NC, BR, DP), lambda i: (0, i, 0)),
            pl.BlockSpec((BR, 1), lambda i: (i, 0)),
            pl.BlockSpec((1, DP), lambda i: (0, 0)),
        ],
        out_specs=pl.BlockSpec((BR, DP), lambda i: (i, 0)),
        out_shape=jax.ShapeDtypeStruct((N_NODES, DP), jnp.float32),
    )(u2, sp, dinv, b2p)


def kernel(x, edge_index, W1, b1, W2, b2):
    row = edge_index[0].astype(jnp.int32)
    col = edge_index[1].astype(jnp.int32)
    pad = E_PAD - N_EDGES
    rowp = jnp.concatenate([row, jnp.zeros((pad,), jnp.int32)])
    rowp = rowp.reshape(TOTCHUNK, CHUNK)
    colp = jnp.concatenate([col, jnp.full((pad,), N_NODES, jnp.int32)])
    colp = colp.reshape(TOTCHUNK, CHUNK)

    w1p = jnp.pad(W1, ((0, 0), (0, DP - N_HID)))
    w2p = jnp.pad(W2, ((0, DP - N_HID), (0, DP - N_HID)))
    b1p = jnp.pad(b1, (0, DP - N_HID)).reshape(1, DP)
    b2p = jnp.pad(b2, (0, DP - N_HID)).reshape(1, DP)

    h1 = _ta_call(x, w1p)
    degp = _deg_call(colp).reshape(NC, N_NODES, 16)
    dinv, u1 = _tb_call(h1, degp)
    s1 = _agg_call(u1, rowp, colp).reshape(NC, N_NODES, DP)
    u2 = _t2_call(u1, s1, dinv, b1p, w2p)
    s2 = _agg_call(u2, rowp, colp).reshape(NC, N_NODES, DP)
    z = _t3_call(u2, s2, dinv, b2p)
    return z[:, :N_HID]


# speedup vs baseline: 1.6603x; 1.6591x over previous
import jax
import jax.numpy as jnp
from jax import lax
from jax.experimental import pallas as pl
from jax.experimental.pallas import tpu as pltpu
from jax.experimental.pallas import tpu_sc as plsc

N_NODES = 10000
N_EDGES = 320000
D_FEAT = 128
N_HID = 20
DP = 32

NC = 2
NS = 16
NW = NC * NS
CHUNK = 128
NCHUNK0 = 80
NCHUNK1 = 80
TOTCHUNK = NS * (NCHUNK0 + NCHUNK1)
E_PAD = TOTCHUNK * CHUNK
NCHUNKD = TOTCHUNK // NW
ROWS_PT = N_NODES // NS
S_ROWS = N_NODES + NS
NBUF = 8
LAG = 3
MAXQ = 8

_MESH = plsc.VectorSubcoreMesh(core_axis_name="c", subcore_axis_name="s")
_SC_PARAMS = pltpu.CompilerParams(use_tc_tiling_on_sc=False)


def _guarded(cond, fn):
    if cond is None:
        fn()
    else:
        pl.when(cond)(fn)


def _fill(ref, nrows, ncols, value):
    v = jnp.full((16,), value, jnp.float32)

    def body(r, carry):
        for k in range(ncols // 16):
            ref[r, pl.ds(k * 16, 16)] = v
        return carry

    lax.fori_loop(0, nrows, body, 0)


def _deg_body(cidx_hbm, out_hbm, cidx_v, ones_v, zbuf_v, s_sh, sem):
    c = lax.axis_index("c")
    s = lax.axis_index("s")
    w = c * NS + s
    pltpu.sync_copy(cidx_hbm.at[pl.ds(w * NCHUNKD, NCHUNKD)], cidx_v)
    _fill(ones_v, CHUNK, 16, 1.0)
    _fill(zbuf_v, ROWS_PT, 16, 0.0)
    pltpu.sync_copy(zbuf_v, s_sh.at[pl.ds(s * ROWS_PT, ROWS_PT)])
    plsc.subcore_barrier()

    descs = []
    for j in range(NCHUNKD):
        if j >= MAXQ:
            descs[j - MAXQ].wait()
        descs.append(
            pltpu.async_copy(ones_v, s_sh.at[cidx_v.at[j]], sem, add=True)
        )
    for d in descs[NCHUNKD - MAXQ:]:
        d.wait()
    plsc.subcore_barrier()
    pltpu.sync_copy(
        s_sh.at[pl.ds(s * ROWS_PT, ROWS_PT)],
        out_hbm.at[c, s],
    )


_deg_call = pl.kernel(
    _deg_body,
    out_type=jax.ShapeDtypeStruct((NC, NS, ROWS_PT, 16), jnp.float32),
    mesh=_MESH,
    scratch_types=[
        pltpu.VMEM((NCHUNKD, CHUNK), jnp.int32),
        pltpu.VMEM((CHUNK, 16), jnp.float32),
        pltpu.VMEM((ROWS_PT, 16), jnp.float32),
        pltpu.VMEM_SHARED((S_ROWS, 16), jnp.float32),
        pltpu.SemaphoreType.DMA,
    ],
    compiler_params=_SC_PARAMS,
)


def _agg_body(u_hbm, ridx_hbm, cidx_hbm, out_hbm,
              ridx_v, cidx_v, gbuf_v, zbuf_v, s_sh, u_sh, *sems):
    gsems = sems[:NBUF]
    ssems = sems[NBUF:]
    c = lax.axis_index("c")
    s = lax.axis_index("s")
    base = jnp.where(c == 0, s * NCHUNK0, NS * NCHUNK0 + s * NCHUNK1)
    limit = jnp.where(c == 0, NCHUNK0, NCHUNK1)
    nx = NCHUNK0 - NCHUNK1
    pltpu.sync_copy(ridx_hbm.at[pl.ds(base, NCHUNK1)], ridx_v.at[pl.ds(0, NCHUNK1)])
    pltpu.sync_copy(cidx_hbm.at[pl.ds(base, NCHUNK1)], cidx_v.at[pl.ds(0, NCHUNK1)])
    if nx:
        @pl.when(c == 0)
        def _():
            pltpu.sync_copy(ridx_hbm.at[pl.ds(base + NCHUNK1, nx)],
                            ridx_v.at[pl.ds(NCHUNK1, nx)])
            pltpu.sync_copy(cidx_hbm.at[pl.ds(base + NCHUNK1, nx)],
                            cidx_v.at[pl.ds(NCHUNK1, nx)])

    _fill(zbuf_v, ROWS_PT, DP, 0.0)
    pltpu.sync_copy(zbuf_v, s_sh.at[pl.ds(s * ROWS_PT, ROWS_PT)])

    @pl.when(s == 0)
    def _():
        pltpu.sync_copy(u_hbm, u_sh)

    plsc.subcore_barrier()

    def G(j):
        return None if j < NCHUNK1 else (c == 0)

    def fire_gather(j):
        b = j % NBUF
        pltpu.async_copy(u_sh.at[ridx_v.at[j]], gbuf_v.at[b], gsems[b])

    def wait_gather(j):
        b = j % NBUF
        pltpu.make_async_copy(u_sh.at[ridx_v.at[j]], gbuf_v.at[b],
                              gsems[b]).wait()

    def fire_scatter(j):
        b = j % NBUF
        pltpu.async_copy(gbuf_v.at[b], s_sh.at[cidx_v.at[j]], ssems[b],
                         add=True)

    def wait_scatter(j):
        b = j % NBUF
        pltpu.make_async_copy(gbuf_v.at[b], s_sh.at[cidx_v.at[j]],
                              ssems[b]).wait()

    for j in range(min(NBUF, NCHUNK0)):
        _guarded(G(j), lambda j=j: fire_gather(j))
    waited = set()
    for j in range(NCHUNK0):
        def step(j=j):
            wait_gather(j)
            fire_scatter(j)
        _guarded(G(j), step)
        k = j - LAG
        if k >= 0 and k + NBUF < NCHUNK0:
            _guarded(G(k), lambda k=k: wait_scatter(k))
            waited.add(k)
            _guarded(G(k + NBUF), lambda k=k: fire_gather(k + NBUF))
    for k in range(NCHUNK0):
        if k not in waited:
            _guarded(G(k), lambda k=k: wait_scatter(k))
    plsc.subcore_barrier()
    pltpu.sync_copy(
        s_sh.at[pl.ds(s * ROWS_PT, ROWS_PT)],
        out_hbm.at[c, s],
    )


_agg_call = pl.kernel(
    _agg_body,
    out_type=jax.ShapeDtypeStruct((NC, NS, ROWS_PT, DP), jnp.float32),
    mesh=_MESH,
    scratch_types=[
        pltpu.VMEM((NCHUNK0, CHUNK), jnp.int32),
        pltpu.VMEM((NCHUNK0, CHUNK), jnp.int32),
        pltpu.VMEM((NBUF, CHUNK, DP), jnp.float32),
        pltpu.VMEM((ROWS_PT, DP), jnp.float32),
        pltpu.VMEM_SHARED((S_ROWS, DP), jnp.float32),
        pltpu.VMEM_SHARED((N_NODES, DP), jnp.float32),
    ] + [pltpu.SemaphoreType.DMA] * (2 * NBUF),
    compiler_params=_SC_PARAMS,
)

BR = 1000
GRID = N_NODES // BR


def _ta_body(x_ref, w1_ref, h1_ref):
    h1_ref[...] = jnp.dot(x_ref[...], w1_ref[...],
                          preferred_element_type=jnp.float32)


def _tb_body(h1_ref, degp_ref, dinv_ref, u1_ref):
    a = degp_ref[...]
    deg = a[0, :, 0:1] + a[1, :, 0:1] + 1.0
    dinv = lax.rsqrt(deg)
    dinv_ref[...] = dinv
    u1_ref[...] = dinv * h1_ref[...]


def _t2_body(u1_ref, sp_ref, dinv_ref, b1_ref, w2_ref, u2_ref):
    sp = sp_ref[...]
    dinv = dinv_ref[...]
    t = dinv * (u1_ref[...] + sp[0] + sp[1]) + b1_ref[...]
    out1 = jnp.maximum(t, 0.0)
    u2_ref[...] = dinv * jnp.dot(out1, w2_ref[...],
                                 preferred_element_type=jnp.float32)


def _t3_body(u2_ref, sp_ref, dinv_ref, b2_ref, out_ref):
    sp = sp_ref[...]
    z = dinv_ref[...] * (u2_ref[...] + sp[0] + sp[1]) + b2_ref[...]
    z = jnp.maximum(z, 0.0)
    col = lax.broadcasted_iota(jnp.int32, z.shape, 1)
    valid = col < N_HID
    zm = jnp.where(valid, z, -jnp.inf)
    m = jnp.max(zm, axis=1, keepdims=True)
    e = jnp.where(valid, jnp.exp(z - m), 0.0)
    lse = m + jnp.log(jnp.sum(e, axis=1, keepdims=True))
    out_ref[...] = z - lse


def _ta_call(x, w1p):
    return pl.pallas_call(
        _ta_body,
        grid=(GRID,),
        in_specs=[
            pl.BlockSpec((BR, D_FEAT), lambda i: (i, 0)),
            pl.BlockSpec((D_FEAT, DP), lambda i: (0, 0)),
        ],
        out_specs=pl.BlockSpec((BR, DP), lambda i: (i, 0)),
        out_shape=jax.ShapeDtypeStruct((N_NODES, DP), jnp.float32),
    )(x, w1p)


def _tb_call(h1, degp):
    return pl.pallas_call(
        _tb_body,
        grid=(GRID,),
        in_specs=[
            pl.BlockSpec((BR, DP), lambda i: (i, 0)),
            pl.BlockSpec((NC, BR, 16), lambda i: (0, i, 0)),
        ],
        out_specs=[
            pl.BlockSpec((BR, 1), lambda i: (i, 0)),
            pl.BlockSpec((BR, DP), lambda i: (i, 0)),
        ],
        out_shape=[
            jax.ShapeDtypeStruct((N_NODES, 1), jnp.float32),
            jax.ShapeDtypeStruct((N_NODES, DP), jnp.float32),
        ],
    )(h1, degp)


def _t2_call(u1, sp, dinv, b1p, w2p):
    return pl.pallas_call(
        _t2_body,
        grid=(GRID,),
        in_specs=[
            pl.BlockSpec((BR, DP), lambda i: (i, 0)),
            pl.BlockSpec((NC, BR, DP), lambda i: (0, i, 0)),
            pl.BlockSpec((BR, 1), lambda i: (i, 0)),
            pl.BlockSpec((1, DP), lambda i: (0, 0)),
            pl.BlockSpec((DP, DP), lambda i: (0, 0)),
        ],
        out_specs=pl.BlockSpec((BR, DP), lambda i: (i, 0)),
        out_shape=jax.ShapeDtypeStruct((N_NODES, DP), jnp.float32),
    )(u1, sp, dinv, b1p, w2p)


def _t3_call(u2, sp, dinv, b2p):
    return pl.pallas_call(
        _t3_body,
        grid=(GRID,),
        in_specs=[
            pl.BlockSpec((BR, DP), lambda i: (i, 0)),
            pl.BlockSpec((NC, BR, DP), lambda i: (0, i, 0)),
            pl.BlockSpec((BR, 1), lambda i: (i, 0)),
            pl.BlockSpec((1, DP), lambda i: (0, 0)),
        ],
        out_specs=pl.BlockSpec((BR, DP), lambda i: (i, 0)),
        out_shape=jax.ShapeDtypeStruct((N_NODES, DP), jnp.float32),
    )(u2, sp, dinv, b2p)


def kernel(x, edge_index, W1, b1, W2, b2):
    row = edge_index[0].astype(jnp.int32)
    col = edge_index[1].astype(jnp.int32)
    pad = E_PAD - N_EDGES
    rowp = jnp.concatenate([row, jnp.zeros((pad,), jnp.int32)])
    rowp = rowp.reshape(TOTCHUNK, CHUNK)
    colp = jnp.concatenate([col, jnp.full((pad,), N_NODES, jnp.int32)])
    colp = colp.reshape(TOTCHUNK, CHUNK)

    w1p = jnp.pad(W1, ((0, 0), (0, DP - N_HID)))
    w2p = jnp.pad(W2, ((0, DP - N_HID), (0, DP - N_HID)))
    b1p = jnp.pad(b1, (0, DP - N_HID)).reshape(1, DP)
    b2p = jnp.pad(b2, (0, DP - N_HID)).reshape(1, DP)

    h1 = _ta_call(x, w1p)
    degp = _deg_call(colp).reshape(NC, N_NODES, 16)
    dinv, u1 = _tb_call(h1, degp)
    s1 = _agg_call(u1, rowp, colp).reshape(NC, N_NODES, DP)
    u2 = _t2_call(u1, s1, dinv, b1p, w2p)
    s2 = _agg_call(u2, rowp, colp).reshape(NC, N_NODES, DP)
    z = _t3_call(u2, s2, dinv, b2p)
    return z[:, :N_HID]


# speedup vs baseline: 1.6787x; 1.0111x over previous
import jax
import jax.numpy as jnp
from jax import lax
from jax.experimental import pallas as pl
from jax.experimental.pallas import tpu as pltpu
from jax.experimental.pallas import tpu_sc as plsc

N_NODES = 10000
N_EDGES = 320000
D_FEAT = 128
N_HID = 20
DP = 32

NC = 2
NS = 16
NW = NC * NS
CHUNK = 128
NCHUNK0 = 84
NCHUNK1 = 76
TOTCHUNK = NS * (NCHUNK0 + NCHUNK1)
E_PAD = TOTCHUNK * CHUNK
NCHUNKD = TOTCHUNK // NW
ROWS_PT = N_NODES // NS
S_ROWS = N_NODES + NS
NBUF = 8
LAG = 3
MAXQ = 8

_MESH = plsc.VectorSubcoreMesh(core_axis_name="c", subcore_axis_name="s")
_SC_PARAMS = pltpu.CompilerParams(use_tc_tiling_on_sc=False)


def _guarded(cond, fn):
    if cond is None:
        fn()
    else:
        pl.when(cond)(fn)


def _fill(ref, nrows, ncols, value):
    v = jnp.full((16,), value, jnp.float32)

    def body(r, carry):
        for k in range(ncols // 16):
            ref[r, pl.ds(k * 16, 16)] = v
        return carry

    lax.fori_loop(0, nrows, body, 0)


def _deg_body(cidx_hbm, out_hbm, cidx_v, ones_v, zbuf_v, s_sh, sem):
    c = lax.axis_index("c")
    s = lax.axis_index("s")
    w = c * NS + s
    pltpu.sync_copy(cidx_hbm.at[pl.ds(w * NCHUNKD, NCHUNKD)], cidx_v)
    _fill(ones_v, CHUNK, 16, 1.0)
    _fill(zbuf_v, ROWS_PT, 16, 0.0)
    pltpu.sync_copy(zbuf_v, s_sh.at[pl.ds(s * ROWS_PT, ROWS_PT)])
    plsc.subcore_barrier()

    descs = []
    for j in range(NCHUNKD):
        if j >= MAXQ:
            descs[j - MAXQ].wait()
        descs.append(
            pltpu.async_copy(ones_v, s_sh.at[cidx_v.at[j]], sem, add=True)
        )
    for d in descs[NCHUNKD - MAXQ:]:
        d.wait()
    plsc.subcore_barrier()
    pltpu.sync_copy(
        s_sh.at[pl.ds(s * ROWS_PT, ROWS_PT)],
        out_hbm.at[c, s],
    )


_deg_call = pl.kernel(
    _deg_body,
    out_type=jax.ShapeDtypeStruct((NC, NS, ROWS_PT, 16), jnp.float32),
    mesh=_MESH,
    scratch_types=[
        pltpu.VMEM((NCHUNKD, CHUNK), jnp.int32),
        pltpu.VMEM((CHUNK, 16), jnp.float32),
        pltpu.VMEM((ROWS_PT, 16), jnp.float32),
        pltpu.VMEM_SHARED((S_ROWS, 16), jnp.float32),
        pltpu.SemaphoreType.DMA,
    ],
    compiler_params=_SC_PARAMS,
)


def _agg_body(u_hbm, ridx_hbm, cidx_hbm, out_hbm,
              ridx_v, cidx_v, gbuf_v, zbuf_v, s_sh, u_sh, *sems):
    gsems = sems[:NBUF]
    ssems = sems[NBUF:]
    c = lax.axis_index("c")
    s = lax.axis_index("s")
    base = jnp.where(c == 0, s * NCHUNK0, NS * NCHUNK0 + s * NCHUNK1)
    nx = NCHUNK0 - NCHUNK1
    pltpu.sync_copy(ridx_hbm.at[pl.ds(base, NCHUNK1)], ridx_v.at[pl.ds(0, NCHUNK1)])
    pltpu.sync_copy(cidx_hbm.at[pl.ds(base, NCHUNK1)], cidx_v.at[pl.ds(0, NCHUNK1)])
    if nx:
        @pl.when(c == 0)
        def _():
            pltpu.sync_copy(ridx_hbm.at[pl.ds(base + NCHUNK1, nx)],
                            ridx_v.at[pl.ds(NCHUNK1, nx)])
            pltpu.sync_copy(cidx_hbm.at[pl.ds(base + NCHUNK1, nx)],
                            cidx_v.at[pl.ds(NCHUNK1, nx)])

    _fill(zbuf_v, ROWS_PT, DP, 0.0)
    pltpu.sync_copy(zbuf_v, s_sh.at[pl.ds(s * ROWS_PT, ROWS_PT)])

    @pl.when(s == 0)
    def _():
        pltpu.sync_copy(u_hbm, u_sh)

    plsc.subcore_barrier()

    def G(j):
        return None if j < NCHUNK1 else (c == 0)

    def fire_gather(j):
        b = j % NBUF
        pltpu.async_copy(u_sh.at[ridx_v.at[j]], gbuf_v.at[b], gsems[b])

    def wait_gather(j):
        b = j % NBUF
        pltpu.make_async_copy(u_sh.at[ridx_v.at[j]], gbuf_v.at[b],
                              gsems[b]).wait()

    def fire_scatter(j):
        b = j % NBUF
        pltpu.async_copy(gbuf_v.at[b], s_sh.at[cidx_v.at[j]], ssems[b],
                         add=True)

    def wait_scatter(j):
        b = j % NBUF
        pltpu.make_async_copy(gbuf_v.at[b], s_sh.at[cidx_v.at[j]],
                              ssems[b]).wait()

    for j in range(min(NBUF, NCHUNK0)):
        _guarded(G(j), lambda j=j: fire_gather(j))
    waited = set()
    for j in range(NCHUNK0):
        def step(j=j):
            wait_gather(j)
            fire_scatter(j)
        _guarded(G(j), step)
        k = j - LAG
        if k >= 0 and k + NBUF < NCHUNK0:
            _guarded(G(k), lambda k=k: wait_scatter(k))
            waited.add(k)
            _guarded(G(k + NBUF), lambda k=k: fire_gather(k + NBUF))
    for k in range(NCHUNK0):
        if k not in waited:
            _guarded(G(k), lambda k=k: wait_scatter(k))
    plsc.subcore_barrier()
    pltpu.sync_copy(
        s_sh.at[pl.ds(s * ROWS_PT, ROWS_PT)],
        out_hbm.at[c, s],
    )


_agg_call = pl.kernel(
    _agg_body,
    out_type=jax.ShapeDtypeStruct((NC, NS, ROWS_PT, DP), jnp.float32),
    mesh=_MESH,
    scratch_types=[
        pltpu.VMEM((NCHUNK0, CHUNK), jnp.int32),
        pltpu.VMEM((NCHUNK0, CHUNK), jnp.int32),
        pltpu.VMEM((NBUF, CHUNK, DP), jnp.float32),
        pltpu.VMEM((ROWS_PT, DP), jnp.float32),
        pltpu.VMEM_SHARED((S_ROWS, DP), jnp.float32),
        pltpu.VMEM_SHARED((N_NODES, DP), jnp.float32),
    ] + [pltpu.SemaphoreType.DMA] * (2 * NBUF),
    compiler_params=_SC_PARAMS,
)

BR = 1000
GRID = N_NODES // BR


def _ta_body(x_ref, w1_ref, h1_ref):
    h1_ref[...] = jnp.dot(x_ref[...], w1_ref[...],
                          preferred_element_type=jnp.float32)


def _tb_body(h1_ref, degp_ref, dinv_ref, u1_ref):
    a = degp_ref[...]
    deg = a[0, :, 0:1] + a[1, :, 0:1] + 1.0
    dinv = lax.rsqrt(deg)
    dinv_ref[...] = dinv
    u1_ref[...] = dinv * h1_ref[...]


def _t2_body(u1_ref, sp_ref, dinv_ref, b1_ref, w2_ref, u2_ref):
    sp = sp_ref[...]
    dinv = dinv_ref[...]
    t = dinv * (u1_ref[...] + sp[0] + sp[1]) + b1_ref[...]
    out1 = jnp.maximum(t, 0.0)
    u2_ref[...] = dinv * jnp.dot(out1, w2_ref[...],
                                 preferred_element_type=jnp.float32)


def _t3_body(u2_ref, sp_ref, dinv_ref, b2_ref, out_ref):
    sp = sp_ref[...]
    z = dinv_ref[...] * (u2_ref[...] + sp[0] + sp[1]) + b2_ref[...]
    z = jnp.maximum(z, 0.0)
    col = lax.broadcasted_iota(jnp.int32, z.shape, 1)
    valid = col < N_HID
    zm = jnp.where(valid, z, -jnp.inf)
    m = jnp.max(zm, axis=1, keepdims=True)
    e = jnp.where(valid, jnp.exp(z - m), 0.0)
    lse = m + jnp.log(jnp.sum(e, axis=1, keepdims=True))
    out_ref[...] = z - lse


def _ta_call(x, w1p):
    return pl.pallas_call(
        _ta_body,
        grid=(GRID,),
        in_specs=[
            pl.BlockSpec((BR, D_FEAT), lambda i: (i, 0)),
            pl.BlockSpec((D_FEAT, DP), lambda i: (0, 0)),
        ],
        out_specs=pl.BlockSpec((BR, DP), lambda i: (i, 0)),
        out_shape=jax.ShapeDtypeStruct((N_NODES, DP), jnp.float32),
    )(x, w1p)


def _tb_call(h1, degp):
    return pl.pallas_call(
        _tb_body,
        grid=(GRID,),
        in_specs=[
            pl.BlockSpec((BR, DP), lambda i: (i, 0)),
            pl.BlockSpec((NC, BR, 16), lambda i: (0, i, 0)),
        ],
        out_specs=[
            pl.BlockSpec((BR, 1), lambda i: (i, 0)),
            pl.BlockSpec((BR, DP), lambda i: (i, 0)),
        ],
        out_shape=[
            jax.ShapeDtypeStruct((N_NODES, 1), jnp.float32),
            jax.ShapeDtypeStruct((N_NODES, DP), jnp.float32),
        ],
    )(h1, degp)


def _t2_call(u1, sp, dinv, b1p, w2p):
    return pl.pallas_call(
        _t2_body,
        grid=(GRID,),
        in_specs=[
            pl.BlockSpec((BR, DP), lambda i: (i, 0)),
            pl.BlockSpec((NC, BR, DP), lambda i: (0, i, 0)),
            pl.BlockSpec((BR, 1), lambda i: (i, 0)),
            pl.BlockSpec((1, DP), lambda i: (0, 0)),
            pl.BlockSpec((DP, DP), lambda i: (0, 0)),
        ],
        out_specs=pl.BlockSpec((BR, DP), lambda i: (i, 0)),
        out_shape=jax.ShapeDtypeStruct((N_NODES, DP), jnp.float32),
    )(u1, sp, dinv, b1p, w2p)


def _t3_call(u2, sp, dinv, b2p):
    return pl.pallas_call(
        _t3_body,
        grid=(GRID,),
        in_specs=[
            pl.BlockSpec((BR, DP), lambda i: (i, 0)),
            pl.BlockSpec((NC, BR, DP), lambda i: (0, i, 0)),
            pl.BlockSpec((BR, 1), lambda i: (i, 0)),
            pl.BlockSpec((1, DP), lambda i: (0, 0)),
        ],
        out_specs=pl.BlockSpec((BR, DP), lambda i: (i, 0)),
        out_shape=jax.ShapeDtypeStruct((N_NODES, DP), jnp.float32),
    )(u2, sp, dinv, b2p)


def kernel(x, edge_index, W1, b1, W2, b2):
    row = edge_index[0].astype(jnp.int32)
    col = edge_index[1].astype(jnp.int32)
    pad = E_PAD - N_EDGES
    rowp = jnp.concatenate([row, jnp.zeros((pad,), jnp.int32)])
    rowp = rowp.reshape(TOTCHUNK, CHUNK)
    colp = jnp.concatenate([col, jnp.full((pad,), N_NODES, jnp.int32)])
    colp = colp.reshape(TOTCHUNK, CHUNK)

    w1p = jnp.pad(W1, ((0, 0), (0, DP - N_HID)))
    w2p = jnp.pad(W2, ((0, DP - N_HID), (0, DP - N_HID)))
    b1p = jnp.pad(b1, (0, DP - N_HID)).reshape(1, DP)
    b2p = jnp.pad(b2, (0, DP - N_HID)).reshape(1, DP)

    h1 = _ta_call(x, w1p)
    degp = _deg_call(colp).reshape(NC, N_NODES, 16)
    dinv, u1 = _tb_call(h1, degp)
    s1 = _agg_call(u1, rowp, colp).reshape(NC, N_NODES, DP)
    u2 = _t2_call(u1, s1, dinv, b1p, w2p)
    s2 = _agg_call(u2, rowp, colp).reshape(NC, N_NODES, DP)
    z = _t3_call(u2, s2, dinv, b2p)
    return z[:, :N_HID]
